# Initial kernel scaffold; baseline (speedup 1.0000x reference)
import functools

import jax
import jax.numpy as jnp
from jax.experimental import pallas as pl

N_ROWS = 320000
DIM = 128
NSEG = 16
BLK = 2560
NBLK = N_ROWS // BLK


def _pass1_body(x_ref, b_ref, wv_ref, bv_ref, wu_ref, bu_ref, wa_ref,
                a_ref, s_ref, d_ref, m_ref):
    i = pl.program_id(0)
    nb = pl.num_programs(0)

    @pl.when(i == 0)
    def _init():
        s_ref[...] = jnp.zeros_like(s_ref)
        d_ref[...] = jnp.zeros_like(d_ref)
        m_ref[...] = jnp.full_like(m_ref, -jnp.inf)

    x = x_ref[...]
    xb = x.astype(jnp.bfloat16)
    hv = jax.lax.dot_general(xb, wv_ref[...], (((1,), (0,)), ((), ())),
                             preferred_element_type=jnp.float32)
    hu = jax.lax.dot_general(xb, wu_ref[...], (((1,), (0,)), ((), ())),
                             preferred_element_type=jnp.float32)
    gate = jnp.tanh(hv + bv_ref[...]) * jax.nn.sigmoid(hu + bu_ref[...])
    a_col = jax.lax.dot_general(gate, wa_ref[...], (((1,), (0,)), ((), ())),
                                preferred_element_type=jnp.float32)
    a_ref[...] = a_col

    m_old = m_ref[...]
    m_new = jnp.maximum(m_old, jnp.max(a_col, keepdims=True))
    alpha = jnp.exp(m_old - m_new)
    e_col = jnp.exp(a_col - m_new)

    seg = jax.lax.broadcasted_iota(jnp.int32, (1, NSEG), 1)
    pw = jnp.where(b_ref[...] == seg, e_col, 0.0)
    contrib = jax.lax.dot_general(pw, x, (((0,), (0,)), ((), ())),
                                  preferred_element_type=jnp.float32)
    ones = jnp.ones((BLK, 1), jnp.float32)
    d_contrib = jax.lax.dot_general(pw, ones, (((0,), (0,)), ((), ())),
                                    preferred_element_type=jnp.float32)

    s_ref[...] = s_ref[...] * alpha + contrib
    d_ref[...] = d_ref[...] * alpha + d_contrib
    m_ref[...] = m_new

    @pl.when(i == nb - 1)
    def _finalize():
        d = d_ref[...]
        dinv = jnp.where(d > 0, 1.0 / d, 0.0)
        s_ref[...] = s_ref[...] * dinv


def _pass2_body(a_ref, b_ref, d_ref, m_ref, attn_ref):
    e = jnp.exp(a_ref[...] - m_ref[...])
    bm = b_ref[...]
    acc = jnp.zeros_like(e)
    for s in range(NSEG):
        acc = jnp.where(bm == s, 1.0 / d_ref[s:s + 1, :], acc)
    attn_ref[...] = e * acc


@jax.jit
def kernel(x, batch, W_V, b_V, W_U, b_U, w_att):
    batch = batch.astype(jnp.int32)
    batch_col = batch.reshape(N_ROWS, 1)

    a_flat, slide, denom, gmax = pl.pallas_call(
        _pass1_body,
        grid=(NBLK,),
        in_specs=[
            pl.BlockSpec((BLK, DIM), lambda i: (i, 0)),
            pl.BlockSpec((BLK, 1), lambda i: (i, 0)),
            pl.BlockSpec((DIM, DIM), lambda i: (0, 0)),
            pl.BlockSpec((1, DIM), lambda i: (0, 0)),
            pl.BlockSpec((DIM, DIM), lambda i: (0, 0)),
            pl.BlockSpec((1, DIM), lambda i: (0, 0)),
            pl.BlockSpec((DIM, 1), lambda i: (0, 0)),
        ],
        out_specs=[
            pl.BlockSpec((BLK, 1), lambda i: (i, 0)),
            pl.BlockSpec((NSEG, DIM), lambda i: (0, 0)),
            pl.BlockSpec((NSEG, 1), lambda i: (0, 0)),
            pl.BlockSpec((1, 1), lambda i: (0, 0)),
        ],
        out_shape=[
            jax.ShapeDtypeStruct((N_ROWS, 1), jnp.float32),
            jax.ShapeDtypeStruct((NSEG, DIM), jnp.float32),
            jax.ShapeDtypeStruct((NSEG, 1), jnp.float32),
            jax.ShapeDtypeStruct((1, 1), jnp.float32),
        ],
    )(x, batch_col, W_V.astype(jnp.bfloat16), b_V.reshape(1, DIM),
      W_U.astype(jnp.bfloat16), b_U.reshape(1, DIM), w_att)

    rows2 = N_ROWS // DIM
    attn2 = pl.pallas_call(
        _pass2_body,
        in_specs=[
            pl.BlockSpec((rows2, DIM), lambda: (0, 0)),
            pl.BlockSpec((rows2, DIM), lambda: (0, 0)),
            pl.BlockSpec((NSEG, 1), lambda: (0, 0)),
            pl.BlockSpec((1, 1), lambda: (0, 0)),
        ],
        out_specs=pl.BlockSpec((rows2, DIM), lambda: (0, 0)),
        out_shape=jax.ShapeDtypeStruct((rows2, DIM), jnp.float32),
    )(a_flat.reshape(rows2, DIM), batch.reshape(rows2, DIM), denom, gmax)

    return slide, attn2.reshape(N_ROWS, 1)


# speedup vs baseline: 18.6566x; 18.6566x over previous
import functools

import jax
import jax.numpy as jnp
from jax.experimental import pallas as pl

N_ROWS = 320000
DIM = 128
NSEG = 16
BLK = 16000
NBLK = N_ROWS // BLK


def _pass1_body(x_ref, b_ref, wcat_ref, bcat_ref, wa_ref, w1_ref,
                a_ref, s_ref, d_ref):
    i = pl.program_id(0)
    nb = pl.num_programs(0)

    @pl.when(i == 0)
    def _init():
        s_ref[...] = jnp.zeros_like(s_ref)
        d_ref[...] = jnp.zeros_like(d_ref)

    x = x_ref[...]
    xb = x.astype(jnp.bfloat16)
    hv = jax.lax.dot_general(xb, wcat_ref[..., :DIM], (((1,), (0,)), ((), ())),
                             preferred_element_type=jnp.float32)
    hu = jax.lax.dot_general(xb, wcat_ref[..., DIM:], (((1,), (0,)), ((), ())),
                             preferred_element_type=jnp.float32)
    tv = jnp.tanh(hv.astype(jnp.bfloat16) + bcat_ref[..., :DIM])
    tu = jnp.tanh(hu.astype(jnp.bfloat16) + bcat_ref[..., DIM:])
    half = jnp.bfloat16(0.5)
    gate = tv * (half * tu + half)
    a_row = jax.lax.dot_general(wa_ref[...], gate, (((0,), (1,)), ((), ())),
                                preferred_element_type=jnp.float32)
    a_ref[...] = a_row.reshape(1, BLK // DIM, DIM)

    e_row = jnp.exp(a_row - w1_ref[...])

    b_row = b_ref[...].reshape(1, BLK)
    seg = jax.lax.broadcasted_iota(jnp.int32, (NSEG, 1), 0)
    pw = jnp.where(b_row == seg, e_row, 0.0)
    contrib = jax.lax.dot_general(pw.astype(jnp.bfloat16), xb,
                                  (((1,), (0,)), ((), ())),
                                  preferred_element_type=jnp.float32)
    d_contrib = jnp.sum(pw, axis=1, keepdims=True)

    s_ref[...] = s_ref[...] + contrib
    d_ref[...] = d_ref[...] + d_contrib

    @pl.when(i == nb - 1)
    def _finalize():
        d = d_ref[...]
        dinv = jnp.where(d > 0, 1.0 / d, 0.0)
        s_ref[...] = s_ref[...] * dinv


def _pass2_body(a_ref, b_ref, d_ref, m_ref, attn_ref):
    e = jnp.exp(a_ref[...] - m_ref[...])
    bm = b_ref[...]
    acc = jnp.zeros_like(e)
    for s in range(NSEG):
        acc = jnp.where(bm == s, 1.0 / d_ref[s:s + 1, :], acc)
    attn_ref[...] = e * acc


@jax.jit
def kernel(x, batch, W_V, b_V, W_U, b_U, w_att):
    rows2 = N_ROWS // DIM
    rblk = BLK // DIM
    batch_i32 = batch.astype(jnp.int32)
    batch3 = batch_i32.reshape(NBLK, rblk, DIM)
    batch2 = batch_i32.reshape(rows2, DIM)

    w1 = jnp.sum(jnp.abs(w_att)).reshape(1, 1)
    wcat = jnp.concatenate([W_V, 0.5 * W_U], axis=1).astype(jnp.bfloat16)
    bcat = jnp.concatenate([b_V, 0.5 * b_U]).reshape(1, 2 * DIM).astype(jnp.bfloat16)
    a_mat, slide, denom = pl.pallas_call(
        _pass1_body,
        grid=(NBLK,),
        in_specs=[
            pl.BlockSpec((BLK, DIM), lambda i: (i, 0)),
            pl.BlockSpec((1, rblk, DIM), lambda i: (i, 0, 0)),
            pl.BlockSpec((DIM, 2 * DIM), lambda i: (0, 0)),
            pl.BlockSpec((1, 2 * DIM), lambda i: (0, 0)),
            pl.BlockSpec((DIM, 1), lambda i: (0, 0)),
            pl.BlockSpec((1, 1), lambda i: (0, 0)),
        ],
        out_specs=[
            pl.BlockSpec((1, rblk, DIM), lambda i: (i, 0, 0)),
            pl.BlockSpec((NSEG, DIM), lambda i: (0, 0)),
            pl.BlockSpec((NSEG, 1), lambda i: (0, 0)),
        ],
        out_shape=[
            jax.ShapeDtypeStruct((NBLK, rblk, DIM), jnp.float32),
            jax.ShapeDtypeStruct((NSEG, DIM), jnp.float32),
            jax.ShapeDtypeStruct((NSEG, 1), jnp.float32),
        ],
    )(x, batch3, wcat, bcat, w_att.astype(jnp.bfloat16), w1)

    attn2 = pl.pallas_call(
        _pass2_body,
        in_specs=[
            pl.BlockSpec((rows2, DIM), lambda: (0, 0)),
            pl.BlockSpec((rows2, DIM), lambda: (0, 0)),
            pl.BlockSpec((NSEG, 1), lambda: (0, 0)),
            pl.BlockSpec((1, 1), lambda: (0, 0)),
        ],
        out_specs=pl.BlockSpec((rows2, DIM), lambda: (0, 0)),
        out_shape=jax.ShapeDtypeStruct((rows2, DIM), jnp.float32),
    )(a_mat.reshape(rows2, DIM), batch2, denom, w1)

    return slide, attn2.reshape(N_ROWS, 1)
